# baseline (speedup 1.0000x reference)
import functools

import jax
import jax.numpy as jnp
from jax import lax
from jax.experimental import pallas as pl
from jax.experimental.pallas import tpu as pltpu
from jax.experimental.pallas import tpu_sc as plsc

D = 1024
N_HEAD = 16
HEAD = 64
E = 8
D_FF = 4096
T = 2048
BT = 256
BM = 64
NT = (2 * T + E * BM) // BM
P = NT * BM
LN_EPS = 1e-5
_SCALE = 1.0 / (HEAD ** 0.5)


def _lnorm(x, g, b):
    mu = jnp.mean(x, axis=-1, keepdims=True)
    var = jnp.mean((x - mu) ** 2, axis=-1, keepdims=True)
    return (x - mu) / jnp.sqrt(var + LN_EPS) * g + b


def _gelu_exact(x):
    u = x * 0.7071067811865476
    s = jnp.sign(u)
    au = jnp.abs(u)
    t = 1.0 / (1.0 + 0.3275911 * au)
    poly = ((((1.061405429 * t - 1.453152027) * t + 1.421413741) * t
             - 0.284496736) * t + 0.254829592) * t
    erf = s * (1.0 - poly * jnp.exp(-au * au))
    return 0.5 * x * (1.0 + erf)


def _k1_body(x_ref, idx_ref, se_ref, wqkvt_ref, g_ref, b_ref, qkvt_ref):
    xv = x_ref[...]
    h = _lnorm(xv, g_ref[...], b_ref[...])
    ids = idx_ref[:, :1]
    m1 = (ids == 1).astype(jnp.float32)
    m2 = (ids == 2).astype(jnp.float32)
    m3 = (ids == 3).astype(jnp.float32)
    m0 = 1.0 - m1 - m2 - m3
    h = (h + m0 * se_ref[0:1, :] + m1 * se_ref[1:2, :]
         + m2 * se_ref[2:3, :] + m3 * se_ref[3:4, :])
    res = lax.dot_general(wqkvt_ref[...], h.astype(jnp.bfloat16),
                          (((0,), (1,)), ((), ())),
                          preferred_element_type=jnp.float32)
    qkvt_ref[...] = res.astype(jnp.bfloat16)


def _k2_body(qt_ref, kt_ref, vt_ref, w1c_ref, ot_ref, w1b_ref):
    w1b_ref[...] = w1c_ref[...].astype(jnp.bfloat16)
    qt = qt_ref[...]
    kt = kt_ref[...]
    st = lax.dot_general(kt, qt, (((0,), (0,)), ((), ())),
                         preferred_element_type=jnp.float32)
    st = st * _SCALE
    st = jnp.clip(st, -30.0, 30.0)
    m = jnp.max(st, axis=0, keepdims=True)
    p = jnp.exp(st - m)
    den = jnp.sum(p, axis=0, keepdims=True) + 1e-10
    numt = lax.dot_general(vt_ref[...], p.astype(jnp.bfloat16),
                           (((1,), (0,)), ((), ())),
                           preferred_element_type=jnp.float32)
    ot_ref[...] = numt / den


def _k3_body(x_ref, attnt_ref, woutb_ref, bout_ref, g2_ref, b2_ref, wg_ref,
             xp_ref, hf_ref, e0_ref, e1_ref, w0_ref, w1_ref):
    proj = lax.dot_general(attnt_ref[...].astype(jnp.bfloat16), woutb_ref[...],
                           (((0,), (0,)), ((), ())),
                           preferred_element_type=jnp.float32)
    xp = x_ref[...] + proj + bout_ref[...]
    xp_ref[...] = xp
    hf = _lnorm(xp, g2_ref[...], b2_ref[...])
    hf_ref[...] = hf
    lg = lax.dot_general(hf.astype(jnp.bfloat16), wg_ref[...],
                         (((1,), (0,)), ((), ())),
                         preferred_element_type=jnp.float32)
    ii = lax.broadcasted_iota(jnp.int32, (BT, E), 1)
    m0 = jnp.max(lg, axis=1, keepdims=True)
    e0 = jnp.min(jnp.where(lg == m0, ii, E), axis=1, keepdims=True)
    neg = jnp.where(ii == e0, -1e30, lg)
    m1 = jnp.max(neg, axis=1, keepdims=True)
    e1 = jnp.min(jnp.where(neg == m1, ii, E), axis=1, keepdims=True)
    z1 = jnp.exp(m1 - m0)
    w0 = 1.0 / (1.0 + z1)
    w1 = z1 / (1.0 + z1)
    e0_ref[...] = jnp.broadcast_to(e0, (BT, 128))
    e1_ref[...] = jnp.broadcast_to(e1, (BT, 128))
    w0_ref[...] = jnp.broadcast_to(w0, (BT, 128))
    w1_ref[...] = jnp.broadcast_to(w1, (BT, 128))


def _k4_body(te_ref, na_ref, hfp_ref, w1_ref, b1_ref, lg_ref, lb_ref, w2_ref,
             b2_ref, ng_ref, nb_ref, o_ref):
    @pl.when(pl.program_id(0) < na_ref[0])
    def _():
        _k4_tile(hfp_ref, w1_ref, b1_ref, lg_ref, lb_ref, w2_ref,
                 b2_ref, ng_ref, nb_ref, o_ref)


def _k4_tile(hfp_ref, w1_ref, b1_ref, lg_ref, lb_ref, w2_ref,
             b2_ref, ng_ref, nb_ref, o_ref):
    H = D_FF // 2
    rows = hfp_ref[...]
    rb = rows.astype(jnp.bfloat16)
    dn = (((1,), (0,)), ((), ()))
    a_l = lax.dot_general(rb, w1_ref[0, :, 0:H], dn,
                          preferred_element_type=jnp.float32) + b1_ref[0, :, 0:H]
    a_r = lax.dot_general(rb, w1_ref[0, :, H:D_FF], dn,
                          preferred_element_type=jnp.float32) + b1_ref[0, :, H:D_FF]
    g_l = _gelu_exact(a_l)
    g_r = _gelu_exact(a_r)
    mu = (jnp.sum(g_l, -1, keepdims=True)
          + jnp.sum(g_r, -1, keepdims=True)) * (1.0 / D_FF)
    var = (jnp.sum((g_l - mu) ** 2, -1, keepdims=True)
           + jnp.sum((g_r - mu) ** 2, -1, keepdims=True)) * (1.0 / D_FF)
    inv = 1.0 / jnp.sqrt(var + LN_EPS)
    n_l = ((g_l - mu) * inv * lg_ref[0, :, 0:H]
           + lb_ref[0, :, 0:H]).astype(jnp.bfloat16)
    n_r = ((g_r - mu) * inv * lg_ref[0, :, H:D_FF]
           + lb_ref[0, :, H:D_FF]).astype(jnp.bfloat16)
    yo = (lax.dot_general(n_l, w2_ref[0, 0:H, :], dn,
                          preferred_element_type=jnp.float32)
          + lax.dot_general(n_r, w2_ref[0, H:D_FF, :], dn,
                            preferred_element_type=jnp.float32)) + b2_ref[0]
    o_ref[...] = _lnorm(rows + yo, ng_ref[0], nb_ref[0])


def _conv_body(i_ref, o_ref):
    o_ref[...] = i_ref[...].astype(jnp.bfloat16)


def _k6_body(xp_ref, g0_ref, g1_ref, w0_ref, w1_ref, o_ref):
    o_ref[...] = (xp_ref[...] + w0_ref[:, :1] * g0_ref[...]
                  + w1_ref[:, :1] * g1_ref[...])


def _sc_info():
    try:
        return plsc.get_sparse_core_info()
    except Exception:
        class _I:
            num_cores = 2
            num_subcores = 16
        return _I()


_SC_INFO = _sc_info()
_NW = _SC_INFO.num_cores * _SC_INFO.num_subcores


def _sc_gather(table, idxv):
    bn = idxv.shape[0]
    d = table.shape[1]
    b_per_w = bn // _NW
    chunks = []
    rem = b_per_w
    while rem > 0:
        c = min(64, rem)
        chunks.append(c)
        rem -= c
    mesh = plsc.VectorSubcoreMesh(core_axis_name="c", subcore_axis_name="s")

    @functools.partial(
        pl.kernel, mesh=mesh,
        out_type=jax.ShapeDtypeStruct((bn, d), table.dtype),
        scratch_types=[pltpu.VMEM((64,), jnp.int32),
                       pltpu.VMEM((64, d), table.dtype),
                       pltpu.SemaphoreType.DMA])
    def k(table_hbm, idx_hbm, out_hbm, idx_v, rows_v, sem):
        wid = lax.axis_index("s") * _SC_INFO.num_cores + lax.axis_index("c")
        base = wid * b_per_w
        for c in chunks:
            pltpu.sync_copy(idx_hbm.at[pl.ds(base, c)],
                            idx_v.at[pl.ds(0, c)])
            pltpu.async_copy(table_hbm.at[idx_v.at[pl.ds(0, c)]],
                             rows_v.at[pl.ds(0, c)], sem).wait()
            pltpu.sync_copy(rows_v.at[pl.ds(0, c)],
                            out_hbm.at[pl.ds(base, c)])
            base += c

    return k(table, idxv)


def kernel(x, idx, struct_embed, W_qkv, W_out, b_out, ln1_g, ln1_b, ln2_g,
           ln2_b, W_gate, W_noise, W1, b1, lnff_g, lnff_b, W2, b2, norm_g,
           norm_b):
    f32 = jnp.float32
    bf16 = jnp.bfloat16
    x2 = x.reshape(T, D)
    idxb = jnp.broadcast_to(idx.reshape(T, 1).astype(jnp.int32), (T, 128))
    se_pad = jnp.zeros((8, D), f32).at[:4].set(struct_embed)
    wqkvt = W_qkv.astype(bf16)
    woutb = W_out.astype(bf16)
    wgb = W_gate.astype(bf16)
    w2b = pl.pallas_call(
        _conv_body,
        grid=(16,),
        in_specs=[pl.BlockSpec((E * D_FF // 16, D), lambda i: (i, 0))],
        out_specs=pl.BlockSpec((E * D_FF // 16, D), lambda i: (i, 0)),
        out_shape=jax.ShapeDtypeStruct((E * D_FF, D), bf16),
    )(W2.reshape(E * D_FF, D)).reshape(E, D_FF, D)
    ln1g = ln1_g.reshape(1, D)
    ln1b = ln1_b.reshape(1, D)
    ln2g = ln2_g.reshape(1, D)
    ln2b = ln2_b.reshape(1, D)
    boutr = b_out.reshape(1, D)
    b1r = b1.reshape(E, 1, D_FF)
    lnffgr = lnff_g.reshape(E, 1, D_FF)
    lnffbr = lnff_b.reshape(E, 1, D_FF)
    b2r = b2.reshape(E, 1, D)
    ngr = norm_g.reshape(E, 1, D)
    nbr = norm_b.reshape(E, 1, D)

    nbt = T // BT
    qkvt = pl.pallas_call(
        _k1_body,
        grid=(nbt,),
        in_specs=[
            pl.BlockSpec((BT, D), lambda i: (i, 0)),
            pl.BlockSpec((BT, 128), lambda i: (i, 0)),
            pl.BlockSpec((8, D), lambda i: (0, 0)),
            pl.BlockSpec((D, 3 * D), lambda i: (0, 0)),
            pl.BlockSpec((1, D), lambda i: (0, 0)),
            pl.BlockSpec((1, D), lambda i: (0, 0)),
        ],
        out_specs=pl.BlockSpec((3 * D, BT), lambda i: (0, i)),
        out_shape=jax.ShapeDtypeStruct((3 * D, T), bf16),
    )(x2, idxb, se_pad, wqkvt, ln1g, ln1b)

    attnt, w1bflat = pl.pallas_call(
        _k2_body,
        grid=(N_HEAD,),
        in_specs=[
            pl.BlockSpec((HEAD, T), lambda h: (h, 0)),
            pl.BlockSpec((HEAD, T), lambda h: (N_HEAD + h, 0)),
            pl.BlockSpec((HEAD, T), lambda h: (2 * N_HEAD + h, 0)),
            pl.BlockSpec((E * D // N_HEAD, D_FF), lambda h: (h, 0)),
        ],
        out_specs=[
            pl.BlockSpec((HEAD, T), lambda h: (h, 0)),
            pl.BlockSpec((E * D // N_HEAD, D_FF), lambda h: (h, 0)),
        ],
        out_shape=[
            jax.ShapeDtypeStruct((D, T), f32),
            jax.ShapeDtypeStruct((E * D, D_FF), bf16),
        ],
    )(qkvt, qkvt, qkvt, W1.reshape(E * D, D_FF))
    w1b = w1bflat.reshape(E, D, D_FF)

    xp, hf, e0b, e1b, w0b, w1b_ = pl.pallas_call(
        _k3_body,
        grid=(nbt,),
        in_specs=[
            pl.BlockSpec((BT, D), lambda i: (i, 0)),
            pl.BlockSpec((D, BT), lambda i: (0, i)),
            pl.BlockSpec((D, D), lambda i: (0, 0)),
            pl.BlockSpec((1, D), lambda i: (0, 0)),
            pl.BlockSpec((1, D), lambda i: (0, 0)),
            pl.BlockSpec((1, D), lambda i: (0, 0)),
            pl.BlockSpec((D, E), lambda i: (0, 0)),
        ],
        out_specs=[
            pl.BlockSpec((BT, D), lambda i: (i, 0)),
            pl.BlockSpec((BT, D), lambda i: (i, 0)),
            pl.BlockSpec((BT, 128), lambda i: (i, 0)),
            pl.BlockSpec((BT, 128), lambda i: (i, 0)),
            pl.BlockSpec((BT, 128), lambda i: (i, 0)),
            pl.BlockSpec((BT, 128), lambda i: (i, 0)),
        ],
        out_shape=[
            jax.ShapeDtypeStruct((T, D), f32),
            jax.ShapeDtypeStruct((T, D), f32),
            jax.ShapeDtypeStruct((T, 128), jnp.int32),
            jax.ShapeDtypeStruct((T, 128), jnp.int32),
            jax.ShapeDtypeStruct((T, 128), f32),
            jax.ShapeDtypeStruct((T, 128), f32),
        ],
    )(x2, attnt, woutb, boutr, ln2g, ln2b, wgb)

    e0 = e0b[:, 0]
    e1 = e1b[:, 0]
    cnt = ((e0[:, None] == jnp.arange(E)[None, :]).astype(jnp.int32)
           + (e1[:, None] == jnp.arange(E)[None, :]).astype(jnp.int32))
    csum = jnp.cumsum(cnt, axis=0)
    excl = csum - cnt
    sizes = csum[-1]
    padded = ((sizes + BM - 1) // BM) * BM
    off = jnp.concatenate([jnp.zeros((1,), jnp.int32),
                           jnp.cumsum(padded)[:-1]])
    tok = jnp.arange(T, dtype=jnp.int32)
    oh0 = (e0[:, None] == jnp.arange(E)[None, :]).astype(jnp.int32)
    oh1 = (e1[:, None] == jnp.arange(E)[None, :]).astype(jnp.int32)
    pos0 = (off[None, :] * oh0 + excl * oh0).sum(axis=1)
    pos1 = (off[None, :] * oh1 + excl * oh1).sum(axis=1)
    row_ids = (jnp.arange(P, dtype=jnp.int32) % T).at[
        jnp.concatenate([pos0, pos1])].set(jnp.concatenate([tok, tok]))
    boundaries = jnp.cumsum(padded)
    tile_starts = jnp.arange(NT, dtype=jnp.int32) * BM
    tile_expert = jnp.minimum(
        (tile_starts[:, None] >= boundaries[None, :]).sum(axis=1),
        E - 1).astype(jnp.int32)

    hf_perm = _sc_gather(hf, row_ids)

    n_active = boundaries[-1:] // BM
    grid_spec = pltpu.PrefetchScalarGridSpec(
        num_scalar_prefetch=2,
        grid=(NT,),
        in_specs=[
            pl.BlockSpec((BM, D), lambda i, te, na: (i, 0)),
            pl.BlockSpec((1, D, D_FF), lambda i, te, na: (te[i], 0, 0)),
            pl.BlockSpec((1, 1, D_FF), lambda i, te, na: (te[i], 0, 0)),
            pl.BlockSpec((1, 1, D_FF), lambda i, te, na: (te[i], 0, 0)),
            pl.BlockSpec((1, 1, D_FF), lambda i, te, na: (te[i], 0, 0)),
            pl.BlockSpec((1, D_FF, D), lambda i, te, na: (te[i], 0, 0)),
            pl.BlockSpec((1, 1, D), lambda i, te, na: (te[i], 0, 0)),
            pl.BlockSpec((1, 1, D), lambda i, te, na: (te[i], 0, 0)),
            pl.BlockSpec((1, 1, D), lambda i, te, na: (te[i], 0, 0)),
        ],
        out_specs=pl.BlockSpec((BM, D), lambda i, te, na: (i, 0)),
    )
    ye_perm = pl.pallas_call(
        _k4_body,
        grid_spec=grid_spec,
        out_shape=jax.ShapeDtypeStruct((P, D), f32),
    )(tile_expert, n_active, hf_perm, w1b, b1r, lnffgr, lnffbr, w2b, b2r,
      ngr, nbr)

    gathered = _sc_gather(ye_perm, jnp.concatenate([pos0, pos1]))

    out = pl.pallas_call(
        _k6_body,
        grid=(nbt,),
        in_specs=[
            pl.BlockSpec((BT, D), lambda i: (i, 0)),
            pl.BlockSpec((BT, D), lambda i: (i, 0)),
            pl.BlockSpec((BT, D), lambda i: (T // BT + i, 0)),
            pl.BlockSpec((BT, 128), lambda i: (i, 0)),
            pl.BlockSpec((BT, 128), lambda i: (i, 0)),
        ],
        out_specs=pl.BlockSpec((BT, D), lambda i: (i, 0)),
        out_shape=jax.ShapeDtypeStruct((T, D), f32),
    )(xp, gathered, gathered, w0b, w1b_)

    return out.reshape(x.shape)


# speedup vs baseline: 1.2226x; 1.2226x over previous
import functools

import jax
import jax.numpy as jnp
from jax import lax
from jax.experimental import pallas as pl
from jax.experimental.pallas import tpu as pltpu
from jax.experimental.pallas import tpu_sc as plsc

D = 1024
N_HEAD = 16
HEAD = 64
E = 8
D_FF = 4096
T = 2048
BT = 256
BM = 128
NT = (2 * T + E * BM) // BM
P = NT * BM
LN_EPS = 1e-5
_SCALE = 1.0 / (HEAD ** 0.5)


def _lnorm(x, g, b):
    mu = jnp.mean(x, axis=-1, keepdims=True)
    var = jnp.mean((x - mu) ** 2, axis=-1, keepdims=True)
    return (x - mu) / jnp.sqrt(var + LN_EPS) * g + b


def _gelu_exact(x):
    u = x * 0.7071067811865476
    s = jnp.sign(u)
    au = jnp.abs(u)
    t = 1.0 / (1.0 + 0.3275911 * au)
    poly = ((((1.061405429 * t - 1.453152027) * t + 1.421413741) * t
             - 0.284496736) * t + 0.254829592) * t
    erf = s * (1.0 - poly * jnp.exp(-au * au))
    return 0.5 * x * (1.0 + erf)


def _k1_body(x_ref, idx_ref, se_ref, wqkvt_ref, g_ref, b_ref, qkvt_ref):
    xv = x_ref[...]
    h = _lnorm(xv, g_ref[...], b_ref[...])
    ids = idx_ref[:, :1]
    m1 = (ids == 1).astype(jnp.float32)
    m2 = (ids == 2).astype(jnp.float32)
    m3 = (ids == 3).astype(jnp.float32)
    m0 = 1.0 - m1 - m2 - m3
    h = (h + m0 * se_ref[0:1, :] + m1 * se_ref[1:2, :]
         + m2 * se_ref[2:3, :] + m3 * se_ref[3:4, :])
    res = lax.dot_general(wqkvt_ref[...], h.astype(jnp.bfloat16),
                          (((0,), (1,)), ((), ())),
                          preferred_element_type=jnp.float32)
    qkvt_ref[...] = res.astype(jnp.bfloat16)


def _k2_body(qt_ref, kt_ref, vt_ref, w1c_ref, ot_ref, w1b_ref):
    w1b_ref[...] = w1c_ref[...].astype(jnp.bfloat16)
    qt = qt_ref[...]
    kt = kt_ref[...]
    st = lax.dot_general(kt, qt, (((0,), (0,)), ((), ())),
                         preferred_element_type=jnp.float32)
    st = st * _SCALE
    st = jnp.clip(st, -30.0, 30.0)
    m = jnp.max(st, axis=0, keepdims=True)
    p = jnp.exp(st - m)
    den = jnp.sum(p, axis=0, keepdims=True) + 1e-10
    numt = lax.dot_general(vt_ref[...], p.astype(jnp.bfloat16),
                           (((1,), (0,)), ((), ())),
                           preferred_element_type=jnp.float32)
    ot_ref[...] = numt / den


def _k3_body(x_ref, attnt_ref, woutb_ref, bout_ref, g2_ref, b2_ref, wg_ref,
             xp_ref, hf_ref, e0_ref, e1_ref, w0_ref, w1_ref):
    proj = lax.dot_general(attnt_ref[...].astype(jnp.bfloat16), woutb_ref[...],
                           (((0,), (0,)), ((), ())),
                           preferred_element_type=jnp.float32)
    xp = x_ref[...] + proj + bout_ref[...]
    xp_ref[...] = xp
    hf = _lnorm(xp, g2_ref[...], b2_ref[...])
    hf_ref[...] = hf
    lg = lax.dot_general(hf.astype(jnp.bfloat16), wg_ref[...],
                         (((1,), (0,)), ((), ())),
                         preferred_element_type=jnp.float32)
    ii = lax.broadcasted_iota(jnp.int32, (BT, E), 1)
    m0 = jnp.max(lg, axis=1, keepdims=True)
    e0 = jnp.min(jnp.where(lg == m0, ii, E), axis=1, keepdims=True)
    neg = jnp.where(ii == e0, -1e30, lg)
    m1 = jnp.max(neg, axis=1, keepdims=True)
    e1 = jnp.min(jnp.where(neg == m1, ii, E), axis=1, keepdims=True)
    z1 = jnp.exp(m1 - m0)
    w0 = 1.0 / (1.0 + z1)
    w1 = z1 / (1.0 + z1)
    e0_ref[...] = jnp.broadcast_to(e0, (BT, 128))
    e1_ref[...] = jnp.broadcast_to(e1, (BT, 128))
    w0_ref[...] = jnp.broadcast_to(w0, (BT, 128))
    w1_ref[...] = jnp.broadcast_to(w1, (BT, 128))


def _k4_body(te_ref, na_ref, hfp_ref, w1_ref, b1_ref, lg_ref, lb_ref, w2_ref,
             b2_ref, ng_ref, nb_ref, o_ref):
    @pl.when(pl.program_id(0) < na_ref[0])
    def _():
        _k4_tile(hfp_ref, w1_ref, b1_ref, lg_ref, lb_ref, w2_ref,
                 b2_ref, ng_ref, nb_ref, o_ref)


def _k4_tile(hfp_ref, w1_ref, b1_ref, lg_ref, lb_ref, w2_ref,
             b2_ref, ng_ref, nb_ref, o_ref):
    H = D_FF // 2
    rows = hfp_ref[...]
    rb = rows.astype(jnp.bfloat16)
    dn = (((1,), (0,)), ((), ()))
    a_l = lax.dot_general(rb, w1_ref[0, :, 0:H], dn,
                          preferred_element_type=jnp.float32) + b1_ref[0, :, 0:H]
    a_r = lax.dot_general(rb, w1_ref[0, :, H:D_FF], dn,
                          preferred_element_type=jnp.float32) + b1_ref[0, :, H:D_FF]
    g_l = _gelu_exact(a_l)
    g_r = _gelu_exact(a_r)
    mu = (jnp.sum(g_l, -1, keepdims=True)
          + jnp.sum(g_r, -1, keepdims=True)) * (1.0 / D_FF)
    var = (jnp.sum((g_l - mu) ** 2, -1, keepdims=True)
           + jnp.sum((g_r - mu) ** 2, -1, keepdims=True)) * (1.0 / D_FF)
    inv = 1.0 / jnp.sqrt(var + LN_EPS)
    n_l = ((g_l - mu) * inv * lg_ref[0, :, 0:H]
           + lb_ref[0, :, 0:H]).astype(jnp.bfloat16)
    n_r = ((g_r - mu) * inv * lg_ref[0, :, H:D_FF]
           + lb_ref[0, :, H:D_FF]).astype(jnp.bfloat16)
    yo = (lax.dot_general(n_l, w2_ref[0, 0:H, :], dn,
                          preferred_element_type=jnp.float32)
          + lax.dot_general(n_r, w2_ref[0, H:D_FF, :], dn,
                            preferred_element_type=jnp.float32)) + b2_ref[0]
    o_ref[...] = _lnorm(rows + yo, ng_ref[0], nb_ref[0])


def _conv_body(i_ref, o_ref):
    o_ref[...] = i_ref[...].astype(jnp.bfloat16)


def _k6_body(xp_ref, g0_ref, g1_ref, w0_ref, w1_ref, o_ref):
    o_ref[...] = (xp_ref[...] + w0_ref[:, :1] * g0_ref[...]
                  + w1_ref[:, :1] * g1_ref[...])


def _sc_info():
    try:
        return plsc.get_sparse_core_info()
    except Exception:
        class _I:
            num_cores = 2
            num_subcores = 16
        return _I()


_SC_INFO = _sc_info()
_NW = _SC_INFO.num_cores * _SC_INFO.num_subcores


def _sc_gather(table, idxv):
    bn = idxv.shape[0]
    d = table.shape[1]
    b_per_w = bn // _NW
    chunks = []
    rem = b_per_w
    while rem > 0:
        c = min(64, rem)
        chunks.append(c)
        rem -= c
    mesh = plsc.VectorSubcoreMesh(core_axis_name="c", subcore_axis_name="s")

    @functools.partial(
        pl.kernel, mesh=mesh,
        out_type=jax.ShapeDtypeStruct((bn, d), table.dtype),
        scratch_types=[pltpu.VMEM((64,), jnp.int32),
                       pltpu.VMEM((64, d), table.dtype),
                       pltpu.SemaphoreType.DMA])
    def k(table_hbm, idx_hbm, out_hbm, idx_v, rows_v, sem):
        wid = lax.axis_index("s") * _SC_INFO.num_cores + lax.axis_index("c")
        base = wid * b_per_w
        for c in chunks:
            pltpu.sync_copy(idx_hbm.at[pl.ds(base, c)],
                            idx_v.at[pl.ds(0, c)])
            pltpu.async_copy(table_hbm.at[idx_v.at[pl.ds(0, c)]],
                             rows_v.at[pl.ds(0, c)], sem).wait()
            pltpu.sync_copy(rows_v.at[pl.ds(0, c)],
                            out_hbm.at[pl.ds(base, c)])
            base += c

    return k(table, idxv)


def kernel(x, idx, struct_embed, W_qkv, W_out, b_out, ln1_g, ln1_b, ln2_g,
           ln2_b, W_gate, W_noise, W1, b1, lnff_g, lnff_b, W2, b2, norm_g,
           norm_b):
    f32 = jnp.float32
    bf16 = jnp.bfloat16
    x2 = x.reshape(T, D)
    idxb = jnp.broadcast_to(idx.reshape(T, 1).astype(jnp.int32), (T, 128))
    se_pad = jnp.zeros((8, D), f32).at[:4].set(struct_embed)
    wqkvt = W_qkv.astype(bf16)
    woutb = W_out.astype(bf16)
    wgb = W_gate.astype(bf16)
    w2b = pl.pallas_call(
        _conv_body,
        grid=(16,),
        in_specs=[pl.BlockSpec((E * D_FF // 16, D), lambda i: (i, 0))],
        out_specs=pl.BlockSpec((E * D_FF // 16, D), lambda i: (i, 0)),
        out_shape=jax.ShapeDtypeStruct((E * D_FF, D), bf16),
    )(W2.reshape(E * D_FF, D)).reshape(E, D_FF, D)
    ln1g = ln1_g.reshape(1, D)
    ln1b = ln1_b.reshape(1, D)
    ln2g = ln2_g.reshape(1, D)
    ln2b = ln2_b.reshape(1, D)
    boutr = b_out.reshape(1, D)
    b1r = b1.reshape(E, 1, D_FF)
    lnffgr = lnff_g.reshape(E, 1, D_FF)
    lnffbr = lnff_b.reshape(E, 1, D_FF)
    b2r = b2.reshape(E, 1, D)
    ngr = norm_g.reshape(E, 1, D)
    nbr = norm_b.reshape(E, 1, D)

    nbt = T // BT
    qkvt = pl.pallas_call(
        _k1_body,
        grid=(nbt,),
        in_specs=[
            pl.BlockSpec((BT, D), lambda i: (i, 0)),
            pl.BlockSpec((BT, 128), lambda i: (i, 0)),
            pl.BlockSpec((8, D), lambda i: (0, 0)),
            pl.BlockSpec((D, 3 * D), lambda i: (0, 0)),
            pl.BlockSpec((1, D), lambda i: (0, 0)),
            pl.BlockSpec((1, D), lambda i: (0, 0)),
        ],
        out_specs=pl.BlockSpec((3 * D, BT), lambda i: (0, i)),
        out_shape=jax.ShapeDtypeStruct((3 * D, T), bf16),
    )(x2, idxb, se_pad, wqkvt, ln1g, ln1b)

    attnt, w1bflat = pl.pallas_call(
        _k2_body,
        grid=(N_HEAD,),
        in_specs=[
            pl.BlockSpec((HEAD, T), lambda h: (h, 0)),
            pl.BlockSpec((HEAD, T), lambda h: (N_HEAD + h, 0)),
            pl.BlockSpec((HEAD, T), lambda h: (2 * N_HEAD + h, 0)),
            pl.BlockSpec((E * D // N_HEAD, D_FF), lambda h: (h, 0)),
        ],
        out_specs=[
            pl.BlockSpec((HEAD, T), lambda h: (h, 0)),
            pl.BlockSpec((E * D // N_HEAD, D_FF), lambda h: (h, 0)),
        ],
        out_shape=[
            jax.ShapeDtypeStruct((D, T), f32),
            jax.ShapeDtypeStruct((E * D, D_FF), bf16),
        ],
    )(qkvt, qkvt, qkvt, W1.reshape(E * D, D_FF))
    w1b = w1bflat.reshape(E, D, D_FF)

    xp, hf, e0b, e1b, w0b, w1b_ = pl.pallas_call(
        _k3_body,
        grid=(nbt,),
        in_specs=[
            pl.BlockSpec((BT, D), lambda i: (i, 0)),
            pl.BlockSpec((D, BT), lambda i: (0, i)),
            pl.BlockSpec((D, D), lambda i: (0, 0)),
            pl.BlockSpec((1, D), lambda i: (0, 0)),
            pl.BlockSpec((1, D), lambda i: (0, 0)),
            pl.BlockSpec((1, D), lambda i: (0, 0)),
            pl.BlockSpec((D, E), lambda i: (0, 0)),
        ],
        out_specs=[
            pl.BlockSpec((BT, D), lambda i: (i, 0)),
            pl.BlockSpec((BT, D), lambda i: (i, 0)),
            pl.BlockSpec((BT, 128), lambda i: (i, 0)),
            pl.BlockSpec((BT, 128), lambda i: (i, 0)),
            pl.BlockSpec((BT, 128), lambda i: (i, 0)),
            pl.BlockSpec((BT, 128), lambda i: (i, 0)),
        ],
        out_shape=[
            jax.ShapeDtypeStruct((T, D), f32),
            jax.ShapeDtypeStruct((T, D), f32),
            jax.ShapeDtypeStruct((T, 128), jnp.int32),
            jax.ShapeDtypeStruct((T, 128), jnp.int32),
            jax.ShapeDtypeStruct((T, 128), f32),
            jax.ShapeDtypeStruct((T, 128), f32),
        ],
    )(x2, attnt, woutb, boutr, ln2g, ln2b, wgb)

    e0 = e0b[:, 0]
    e1 = e1b[:, 0]
    cnt = ((e0[:, None] == jnp.arange(E)[None, :]).astype(jnp.int32)
           + (e1[:, None] == jnp.arange(E)[None, :]).astype(jnp.int32))
    csum = jnp.cumsum(cnt, axis=0)
    excl = csum - cnt
    sizes = csum[-1]
    padded = ((sizes + BM - 1) // BM) * BM
    off = jnp.concatenate([jnp.zeros((1,), jnp.int32),
                           jnp.cumsum(padded)[:-1]])
    tok = jnp.arange(T, dtype=jnp.int32)
    oh0 = (e0[:, None] == jnp.arange(E)[None, :]).astype(jnp.int32)
    oh1 = (e1[:, None] == jnp.arange(E)[None, :]).astype(jnp.int32)
    pos0 = (off[None, :] * oh0 + excl * oh0).sum(axis=1)
    pos1 = (off[None, :] * oh1 + excl * oh1).sum(axis=1)
    row_ids = (jnp.arange(P, dtype=jnp.int32) % T).at[
        jnp.concatenate([pos0, pos1])].set(jnp.concatenate([tok, tok]))
    boundaries = jnp.cumsum(padded)
    tile_starts = jnp.arange(NT, dtype=jnp.int32) * BM
    tile_expert = jnp.minimum(
        (tile_starts[:, None] >= boundaries[None, :]).sum(axis=1),
        E - 1).astype(jnp.int32)

    hf_perm = _sc_gather(hf, row_ids)

    n_active = boundaries[-1:] // BM
    grid_spec = pltpu.PrefetchScalarGridSpec(
        num_scalar_prefetch=2,
        grid=(NT,),
        in_specs=[
            pl.BlockSpec((BM, D), lambda i, te, na: (i, 0)),
            pl.BlockSpec((1, D, D_FF), lambda i, te, na: (te[i], 0, 0)),
            pl.BlockSpec((1, 1, D_FF), lambda i, te, na: (te[i], 0, 0)),
            pl.BlockSpec((1, 1, D_FF), lambda i, te, na: (te[i], 0, 0)),
            pl.BlockSpec((1, 1, D_FF), lambda i, te, na: (te[i], 0, 0)),
            pl.BlockSpec((1, D_FF, D), lambda i, te, na: (te[i], 0, 0)),
            pl.BlockSpec((1, 1, D), lambda i, te, na: (te[i], 0, 0)),
            pl.BlockSpec((1, 1, D), lambda i, te, na: (te[i], 0, 0)),
            pl.BlockSpec((1, 1, D), lambda i, te, na: (te[i], 0, 0)),
        ],
        out_specs=pl.BlockSpec((BM, D), lambda i, te, na: (i, 0)),
    )
    ye_perm = pl.pallas_call(
        _k4_body,
        grid_spec=grid_spec,
        out_shape=jax.ShapeDtypeStruct((P, D), f32),
    )(tile_expert, n_active, hf_perm, w1b, b1r, lnffgr, lnffbr, w2b, b2r,
      ngr, nbr)

    gathered = _sc_gather(ye_perm, jnp.concatenate([pos0, pos1]))

    out = pl.pallas_call(
        _k6_body,
        grid=(nbt,),
        in_specs=[
            pl.BlockSpec((BT, D), lambda i: (i, 0)),
            pl.BlockSpec((BT, D), lambda i: (i, 0)),
            pl.BlockSpec((BT, D), lambda i: (T // BT + i, 0)),
            pl.BlockSpec((BT, 128), lambda i: (i, 0)),
            pl.BlockSpec((BT, 128), lambda i: (i, 0)),
        ],
        out_specs=pl.BlockSpec((BT, D), lambda i: (i, 0)),
        out_shape=jax.ShapeDtypeStruct((T, D), f32),
    )(xp, gathered, gathered, w0b, w1b_)

    return out.reshape(x.shape)
